# baseline (speedup 1.0000x reference)
import functools

import jax
import jax.numpy as jnp
from jax import lax
from jax.experimental import pallas as pl
from jax.experimental.pallas import tpu as pltpu
from jax.experimental.pallas import tpu_sc as plsc

_NUM_CORES = 2
_NUM_SUBCORES = 16
_NUM_WORKERS = _NUM_CORES * _NUM_SUBCORES

_CHUNK = 32
_NBUF = 2


@functools.partial(jax.jit, static_argnames=("batch", "seq", "hidden"))
def _lookup(positions, table, *, batch, seq, hidden):
    n_rows = batch * seq
    stripe = seq // _NUM_WORKERS
    chunks_per_row = stripe // _CHUNK
    n_chunks = batch * chunks_per_row
    n_grp = n_chunks // _NBUF
    mesh = plsc.VectorSubcoreMesh(core_axis_name="c", subcore_axis_name="s")

    @functools.partial(
        pl.kernel,
        mesh=mesh,
        out_type=jax.ShapeDtypeStruct((n_rows, hidden), jnp.float32),
        scratch_types=(
            [pltpu.VMEM((batch, stripe), jnp.int32)]
            + [pltpu.VMEM((_CHUNK, hidden), jnp.float32)] * _NBUF
            + [pltpu.SemaphoreType.DMA] * (2 * _NBUF)
        ),
    )
    def emb_kernel(idx_hbm, table_hbm, out_hbm, idx_v, *bufs):
        rows = bufs[:_NBUF]
        gsem = bufs[_NBUF:2 * _NBUF]
        osem = bufs[2 * _NBUF:]

        wid = lax.axis_index("s") * _NUM_CORES + lax.axis_index("c")
        col0 = wid * stripe

        pltpu.sync_copy(idx_hbm.at[:, pl.ds(col0, stripe)], idx_v)

        def out_off(g):
            return (g // chunks_per_row) * seq + col0 + (g % chunks_per_row) * _CHUNK

        def gather(g, j):
            r = g // chunks_per_row
            c = (g % chunks_per_row) * _CHUNK
            return pltpu.make_async_copy(
                table_hbm.at[idx_v.at[r, pl.ds(c, _CHUNK)]], rows[j], gsem[j])

        def wb(g, j):
            return pltpu.make_async_copy(
                rows[j], out_hbm.at[pl.ds(out_off(g), _CHUNK)], osem[j])

        def body(grp, carry):
            g0 = grp * _NBUF
            for j in range(_NBUF):
                @pl.when(grp >= 1)
                def _wait_prev():
                    wb(g0 + j - _NBUF, j).wait()
                gather(g0 + j, j).start()
            for j in range(_NBUF):
                gather(g0 + j, j).wait()
                wb(g0 + j, j).start()
            return carry

        lax.fori_loop(0, n_grp, body, 0)
        for j in range(_NBUF):
            wb((n_grp - 1) * _NBUF + j, j).wait()

    return emb_kernel(positions, table)


def kernel(positions, pos_embedding):
    b, t = positions.shape
    hidden = pos_embedding.shape[1]
    out = _lookup(positions.astype(jnp.int32), pos_embedding,
                  batch=b, seq=t, hidden=hidden)
    return out.reshape(b, t, hidden)


# speedup vs baseline: 1.0143x; 1.0143x over previous
import functools

import jax
import jax.numpy as jnp
from jax import lax
from jax.experimental import pallas as pl
from jax.experimental.pallas import tpu as pltpu
from jax.experimental.pallas import tpu_sc as plsc

_NUM_CORES = 2
_NUM_SUBCORES = 16
_NUM_WORKERS = _NUM_CORES * _NUM_SUBCORES

_CHUNK = 32
_NBUF = 3


@functools.partial(jax.jit, static_argnames=("batch", "seq", "hidden"))
def _lookup(positions, table, *, batch, seq, hidden):
    n_rows = batch * seq
    stripe = seq // _NUM_WORKERS
    chunks_per_row = stripe // _CHUNK
    n_chunks = batch * chunks_per_row
    mesh = plsc.VectorSubcoreMesh(core_axis_name="c", subcore_axis_name="s")

    @functools.partial(
        pl.kernel,
        mesh=mesh,
        out_type=jax.ShapeDtypeStruct((n_rows, hidden), jnp.float32),
        scratch_types=(
            [pltpu.VMEM((batch, stripe), jnp.int32)]
            + [pltpu.VMEM((_CHUNK, hidden), jnp.float32)] * _NBUF
            + [pltpu.SemaphoreType.DMA] * (2 * _NBUF)
        ),
    )
    def emb_kernel(idx_hbm, table_hbm, out_hbm, idx_v, *bufs):
        rows = bufs[:_NBUF]
        gsem = bufs[_NBUF:2 * _NBUF]
        osem = bufs[2 * _NBUF:]

        wid = lax.axis_index("s") * _NUM_CORES + lax.axis_index("c")
        col0 = wid * stripe

        pltpu.sync_copy(idx_hbm.at[:, pl.ds(col0, stripe)], idx_v)

        gcp = [None] * n_chunks
        ocp = [None] * n_chunks

        def out_off(g):
            r, c = divmod(g, chunks_per_row)
            return r * seq + col0 + c * _CHUNK

        def writeback(g):
            b = g % _NBUF
            gcp[g].wait()
            ocp[g] = pltpu.async_copy(
                rows[b], out_hbm.at[pl.ds(out_off(g), _CHUNK)], osem[b])

        for g in range(n_chunks):
            b = g % _NBUF
            r, c = divmod(g, chunks_per_row)
            if g >= _NBUF:
                ocp[g - _NBUF].wait()
            gcp[g] = pltpu.async_copy(
                table_hbm.at[idx_v.at[r, pl.ds(c * _CHUNK, _CHUNK)]],
                rows[b], gsem[b])
            if g >= 1:
                writeback(g - 1)

        writeback(n_chunks - 1)
        for g in range(max(0, n_chunks - _NBUF), n_chunks):
            ocp[g].wait()

    return emb_kernel(positions, table)


def kernel(positions, pos_embedding):
    b, t = positions.shape
    hidden = pos_embedding.shape[1]
    out = _lookup(positions.astype(jnp.int32), pos_embedding,
                  batch=b, seq=t, hidden=hidden)
    return out.reshape(b, t, hidden)


# speedup vs baseline: 1.0295x; 1.0150x over previous
import functools

import jax
import jax.numpy as jnp
from jax import lax
from jax.experimental import pallas as pl
from jax.experimental.pallas import tpu as pltpu
from jax.experimental.pallas import tpu_sc as plsc

_NUM_CORES = 2
_NUM_SUBCORES = 16
_NUM_WORKERS = _NUM_CORES * _NUM_SUBCORES

_CHUNK = 32
_NBUF = 3


@functools.partial(jax.jit, static_argnames=("batch", "seq", "hidden"))
def _lookup(positions, table, *, batch, seq, hidden):
    n_rows = batch * seq
    stripe = seq // _NUM_WORKERS
    chunks_per_row = stripe // _CHUNK
    n_chunks = batch * chunks_per_row
    mesh = plsc.VectorSubcoreMesh(core_axis_name="c", subcore_axis_name="s")

    @functools.partial(
        pl.kernel,
        mesh=mesh,
        out_type=jax.ShapeDtypeStruct((n_rows, hidden), jnp.float32),
        scratch_types=(
            [pltpu.VMEM((batch, stripe), jnp.int32)]
            + [pltpu.VMEM((_CHUNK, hidden), jnp.float32)] * _NBUF
            + [pltpu.SemaphoreType.DMA] * (2 * _NBUF)
        ),
    )
    def emb_kernel(idx_hbm, table_hbm, out_hbm, idx_v, *bufs):
        rows = bufs[:_NBUF]
        gsem = bufs[_NBUF:2 * _NBUF]
        osem = bufs[2 * _NBUF:]

        wid = lax.axis_index("s") * _NUM_CORES + lax.axis_index("c")
        col0 = wid * stripe

        pltpu.sync_copy(idx_hbm.at[:, pl.ds(col0, stripe)], idx_v)

        gcp = [None] * n_chunks
        ocp = [None] * n_chunks

        def out_off(g):
            r, c = divmod(g, chunks_per_row)
            return r * seq + col0 + c * _CHUNK

        def writeback(g):
            b = g % _NBUF
            gcp[g].wait()
            ocp[g] = pltpu.async_copy(
                rows[b], out_hbm.at[pl.ds(out_off(g), _CHUNK)], osem[b])

        for g in range(n_chunks):
            b = g % _NBUF
            r, c = divmod(g, chunks_per_row)
            if g >= _NBUF:
                ocp[g - _NBUF].wait()
            gcp[g] = pltpu.async_copy(
                table_hbm.at[idx_v.at[r, pl.ds(c * _CHUNK, _CHUNK)]],
                rows[b], gsem[b])
            if g >= 2:
                writeback(g - 2)

        writeback(n_chunks - 2)
        writeback(n_chunks - 1)
        for g in range(max(0, n_chunks - _NBUF), n_chunks):
            ocp[g].wait()

    return emb_kernel(positions, table)


def kernel(positions, pos_embedding):
    b, t = positions.shape
    hidden = pos_embedding.shape[1]
    out = _lookup(positions.astype(jnp.int32), pos_embedding,
                  batch=b, seq=t, hidden=hidden)
    return out.reshape(b, t, hidden)
